# Initial kernel scaffold; baseline (speedup 1.0000x reference)
import functools

import jax
import jax.numpy as jnp
from jax import lax
from jax.experimental import pallas as pl
from jax.experimental.pallas import tpu as pltpu
from jax.experimental.pallas import tpu_sc as plsc

_B, _L, _D = 4096, 200, 64
_N = _B * _L
_NC, _NS = 2, 16
_NW = _NC * _NS
_PER_W = _N // _NW
_C = 512
_CHUNKS = _PER_W // _C
_G = 128
_NG = _C // _G


def _emb_body(ids_hbm, pos_hbm, tok_hbm, ptab_hbm, out_hbm,
              idx_t, idx_p, rows_t, rows_p, sem_t, sem_p):
    wid = lax.axis_index("s") * _NC + lax.axis_index("c")
    row_base = wid * _PER_W
    irow_base = wid * (_PER_W // _G)

    def chunk(ci, carry):
        r0 = irow_base + ci * _NG
        e0 = row_base + ci * _C
        pltpu.sync_copy(ids_hbm.at[pl.ds(r0, _NG)], idx_t)
        pltpu.sync_copy(pos_hbm.at[pl.ds(r0, _NG)], idx_p)
        copies = []
        for m in range(_NG):
            copies.append(pltpu.async_copy(
                tok_hbm.at[idx_t.at[m]], rows_t.at[pl.ds(m * _G, _G)], sem_t))
            copies.append(pltpu.async_copy(
                ptab_hbm.at[idx_p.at[m]], rows_p.at[pl.ds(m * _G, _G)], sem_p))
        for cp in copies:
            cp.wait()

        def fix_group(g, carry):
            m = g // (_G // 16)
            lo = (g % (_G // 16)) * 16
            vt = idx_t[m, pl.ds(lo, 16)]
            vp = idx_p[m, pl.ds(lo, 16)]
            nz = jnp.sum(jnp.where(vt == 0, 1, 0) + jnp.where(vp == 0, 1, 0))

            @pl.when(nz > 0)
            def _():
                for l in range(16):
                    row = m * _G + lo + l
                    st = idx_t[m, lo + l]
                    sp = idx_p[m, lo + l]

                    @pl.when(st == 0)
                    def _():
                        for k in range(4):
                            rows_t[row, pl.ds(k * 16, 16)] = jnp.zeros(
                                (16,), jnp.float32)

                    @pl.when(sp == 0)
                    def _():
                        for k in range(4):
                            rows_p[row, pl.ds(k * 16, 16)] = jnp.zeros(
                                (16,), jnp.float32)
            return carry

        lax.fori_loop(0, _C // 16, fix_group, 0)

        def add_row(j, carry):
            for k in range(4):
                sl = pl.ds(k * 16, 16)
                rows_t[j, sl] = rows_t[j, sl] + rows_p[j, sl]
            return carry

        lax.fori_loop(0, _C, add_row, 0)

        pltpu.sync_copy(rows_t, out_hbm.at[pl.ds(e0, _C)])
        return carry

    lax.fori_loop(0, _CHUNKS, chunk, 0)


_emb = functools.partial(
    pl.kernel,
    out_type=jax.ShapeDtypeStruct((_N, _D), jnp.float32),
    mesh=plsc.VectorSubcoreMesh(core_axis_name="c", subcore_axis_name="s"),
    scratch_types=[
        pltpu.VMEM((_NG, _G), jnp.int32),
        pltpu.VMEM((_NG, _G), jnp.int32),
        pltpu.VMEM((_C, _D), jnp.float32),
        pltpu.VMEM((_C, _D), jnp.float32),
        pltpu.SemaphoreType.DMA,
        pltpu.SemaphoreType.DMA,
    ],
)(_emb_body)


def kernel(ids, pos, tok_table, pos_table):
    ids2 = ids.reshape(_N // _G, _G)
    pos2 = pos.reshape(_N // _G, _G)
    out = _emb(ids2, pos2, tok_table, pos_table)
    return out.reshape(_B, _L, _D)


# speedup vs baseline: 1.7928x; 1.7928x over previous
import functools

import jax
import jax.numpy as jnp
from jax import lax
from jax.experimental import pallas as pl
from jax.experimental.pallas import tpu as pltpu
from jax.experimental.pallas import tpu_sc as plsc

_B, _L, _D = 4096, 200, 64
_N = _B * _L
_NC, _NS = 2, 16
_NW = _NC * _NS
_PER_W = _N // _NW
_C = 512
_CHUNKS = _PER_W // _C
_G = 128
_NG = _C // _G


def _emb_body(ids_hbm, pos_hbm, tok_hbm, ptab_hbm, out_hbm,
              idx_t, idx_p, rows_t, rows_p, sem_t, sem_p):
    wid = lax.axis_index("s") * _NC + lax.axis_index("c")
    row_base = wid * _PER_W
    irow_base = wid * (_PER_W // _G)

    def chunk(ci, carry):
        r0 = irow_base + ci * _NG
        e0 = row_base + ci * _C
        pltpu.sync_copy(ids_hbm.at[pl.ds(r0, _NG)], idx_t)
        pltpu.sync_copy(pos_hbm.at[pl.ds(r0, _NG)], idx_p)
        copies = []
        for m in range(_NG):
            copies.append(pltpu.async_copy(
                tok_hbm.at[idx_t.at[m]], rows_t.at[pl.ds(m * _G, _G)], sem_t))
            copies.append(pltpu.async_copy(
                ptab_hbm.at[idx_p.at[m]], rows_p.at[pl.ds(m * _G, _G)], sem_p))
        for cp in copies:
            cp.wait()

        def grp(g, carry):
            m = g // (_G // 16)
            lo = (g % (_G // 16)) * 16
            vt = idx_t[m, pl.ds(lo, 16)]
            vp = idx_p[m, pl.ds(lo, 16)]
            for l in range(16):
                row = g * 16 + l
                ft = jnp.broadcast_to((vt[l] != 0).astype(jnp.float32), (16,))
                fp = jnp.broadcast_to((vp[l] != 0).astype(jnp.float32), (16,))
                for k in range(4):
                    sl = pl.ds(k * 16, 16)
                    rows_t[row, sl] = (rows_t[row, sl] * ft
                                       + rows_p[row, sl] * fp)
            return carry

        lax.fori_loop(0, _C // 16, grp, 0)

        pltpu.sync_copy(rows_t, out_hbm.at[pl.ds(e0, _C)])
        return carry

    lax.fori_loop(0, _CHUNKS, chunk, 0)


_emb = functools.partial(
    pl.kernel,
    out_type=jax.ShapeDtypeStruct((_N, _D), jnp.float32),
    mesh=plsc.VectorSubcoreMesh(core_axis_name="c", subcore_axis_name="s"),
    compiler_params=pltpu.CompilerParams(use_tc_tiling_on_sc=False),
    scratch_types=[
        pltpu.VMEM((_NG, _G), jnp.int32),
        pltpu.VMEM((_NG, _G), jnp.int32),
        pltpu.VMEM((_C, _D), jnp.float32),
        pltpu.VMEM((_C, _D), jnp.float32),
        pltpu.SemaphoreType.DMA,
        pltpu.SemaphoreType.DMA,
    ],
)(_emb_body)


def kernel(ids, pos, tok_table, pos_table):
    ids2 = ids.reshape(_N // _G, _G)
    pos2 = pos.reshape(_N // _G, _G)
    out = _emb(ids2, pos2, tok_table, pos_table)
    return out.reshape(_B, _L, _D)


# speedup vs baseline: 2.0805x; 1.1605x over previous
import functools

import jax
import jax.numpy as jnp
from jax import lax
from jax.experimental import pallas as pl
from jax.experimental.pallas import tpu as pltpu
from jax.experimental.pallas import tpu_sc as plsc

_B, _L, _D = 4096, 200, 64
_N = _B * _L
_NC, _NS = 2, 16
_NW = _NC * _NS
_PER_W = _N // _NW
_C = 128
_CHUNKS = _PER_W // _C
_NB = 6
_K = _NB - 2


def _emb_body(ids_hbm, pos_hbm, tok_hbm, ptab_hbm, out_hbm,
              idx_t, idx_p, rows_t, rows_p, sem_idx, sem_g, sem_out):
    wid = lax.axis_index("s") * _NC + lax.axis_index("c")
    row_base = wid * _PER_W
    irow_base = wid * _CHUNKS

    def issue_idx(j, b):
        r = irow_base + j
        pltpu.async_copy(ids_hbm.at[pl.ds(r, 1)], idx_t[b], sem_idx[b])
        pltpu.async_copy(pos_hbm.at[pl.ds(r, 1)], idx_p[b], sem_idx[b])

    def wait_idx(b):
        pltpu.make_async_copy(ids_hbm.at[pl.ds(0, 1)], idx_t[b],
                              sem_idx[b]).wait()
        pltpu.make_async_copy(pos_hbm.at[pl.ds(0, 1)], idx_p[b],
                              sem_idx[b]).wait()

    def issue_gather(b):
        pltpu.async_copy(tok_hbm.at[idx_t[b].at[0]], rows_t[b], sem_g[b])
        pltpu.async_copy(ptab_hbm.at[idx_p[b].at[0]], rows_p[b], sem_g[b])

    def wait_gather(b):
        pltpu.make_async_copy(tok_hbm.at[idx_t[b].at[0]], rows_t[b],
                              sem_g[b]).wait()
        pltpu.make_async_copy(ptab_hbm.at[idx_p[b].at[0]], rows_p[b],
                              sem_g[b]).wait()

    def issue_store(j, b):
        e = row_base + j * _C
        pltpu.async_copy(rows_t[b], out_hbm.at[pl.ds(e, _C)], sem_out[b])

    def wait_store(b):
        pltpu.make_async_copy(rows_t[b], out_hbm.at[pl.ds(0, _C)],
                              sem_out[b]).wait()

    for j in range(_NB):
        issue_idx(j, j)
    for j in range(_K):
        wait_idx(j)
        issue_gather(j)

    def step(cg, carry):
        for b in range(_NB):
            ci = cg * _NB + b

            @pl.when(ci < _CHUNKS)
            def _(ci=ci, b=b):
                gj = ci + _K
                gb = (b + _K) % _NB

                @pl.when(gj < _CHUNKS)
                def _():
                    wait_idx(gb)

                    @pl.when(gj >= _NB)
                    def _():
                        wait_store(gb)
                    issue_gather(gb)

                wait_gather(b)

                def grp(g, carry):
                    lo = g * 16
                    vt = idx_t[b][0, pl.ds(lo, 16)]
                    vp = idx_p[b][0, pl.ds(lo, 16)]
                    for l in range(16):
                        row = lo + l
                        ft = jnp.broadcast_to(
                            (vt[l] != 0).astype(jnp.float32), (16,))
                        fp = jnp.broadcast_to(
                            (vp[l] != 0).astype(jnp.float32), (16,))
                        for k in range(4):
                            sl = pl.ds(k * 16, 16)
                            rows_t[b][row, sl] = (rows_t[b][row, sl] * ft
                                                  + rows_p[b][row, sl] * fp)
                    return carry

                lax.fori_loop(0, _C // 16, grp, 0)

                issue_store(ci, b)

                ij = ci + _NB

                @pl.when(ij < _CHUNKS)
                def _():
                    issue_idx(ij, b)
        return carry

    lax.fori_loop(0, (_CHUNKS + _NB - 1) // _NB, step, 0)

    for j in range(_CHUNKS - _NB, _CHUNKS):
        wait_store(j % _NB)


_emb = functools.partial(
    pl.kernel,
    out_type=jax.ShapeDtypeStruct((_N, _D), jnp.float32),
    mesh=plsc.VectorSubcoreMesh(core_axis_name="c", subcore_axis_name="s"),
    compiler_params=pltpu.CompilerParams(use_tc_tiling_on_sc=False),
    scratch_types=[
        [pltpu.VMEM((1, _C), jnp.int32) for _ in range(_NB)],
        [pltpu.VMEM((1, _C), jnp.int32) for _ in range(_NB)],
        [pltpu.VMEM((_C, _D), jnp.float32) for _ in range(_NB)],
        [pltpu.VMEM((_C, _D), jnp.float32) for _ in range(_NB)],
        [pltpu.SemaphoreType.DMA for _ in range(_NB)],
        [pltpu.SemaphoreType.DMA for _ in range(_NB)],
        [pltpu.SemaphoreType.DMA for _ in range(_NB)],
    ],
)(_emb_body)


def kernel(ids, pos, tok_table, pos_table):
    ids2 = ids.reshape(_N // _C, _C)
    pos2 = pos.reshape(_N // _C, _C)
    out = _emb(ids2, pos2, tok_table, pos_table)
    return out.reshape(_B, _L, _D)
